# baseline (speedup 1.0000x reference)
import functools

import jax
import jax.numpy as jnp
from jax import lax
from jax.experimental import pallas as pl
from jax.experimental.pallas import tpu as pltpu
from jax.experimental.pallas import tpu_sc as plsc

_B, _S, _D = 4, 4096, 1024
_BS = 2048

_SC_ROWS = 4096
_NC, _NS = 2, 16
_NW = _NC * _NS
_ROWS_W = _SC_ROWS // _NW
_CH = 16
_NCHUNK = _ROWS_W // _CH
_NBUF = 3
_CD = _CH * _D


def _add_body(x_ref, pe_ref, o_ref):
    o_ref[...] = x_ref[...] + pe_ref[...][None]


def _tc_add(x, pe_table):
    B, S, D = x.shape
    grid = (S // _BS, B)
    return pl.pallas_call(
        _add_body,
        grid=grid,
        in_specs=[
            pl.BlockSpec((1, _BS, D), lambda s, b: (b, s, 0)),
            pl.BlockSpec((_BS, D), lambda s, b: (s, 0)),
        ],
        out_specs=pl.BlockSpec((1, _BS, D), lambda s, b: (b, s, 0)),
        out_shape=jax.ShapeDtypeStruct((B, S, D), x.dtype),
        compiler_params=pltpu.CompilerParams(
            dimension_semantics=("arbitrary", "arbitrary"),
        ),
    )(x, pe_table)


def _sc_copy_body(x_hbm, o_hbm, xbufs, xsems, osems):
    c = lax.axis_index("c")
    s = lax.axis_index("s")
    wid = s * _NC + c
    base = wid * _ROWS_W

    def in_copy(i):
        sl = i % _NBUF
        off = (base + i * _CH) * _D
        return pltpu.make_async_copy(x_hbm.at[pl.ds(off, _CD)], xbufs[sl], xsems[sl])

    def out_copy(i):
        sl = i % _NBUF
        off = (base + i * _CH) * _D
        return pltpu.make_async_copy(xbufs[sl], o_hbm.at[pl.ds(off, _CD)], osems[sl])

    in_copy(0).start()
    in_copy(1).start()
    for i in range(_NCHUNK):
        in_copy(i).wait()
        if i + 2 < _NCHUNK:
            if i >= 1:
                out_copy(i - 1).wait()
            in_copy(i + 2).start()
        out_copy(i).start()
    for i in range(max(0, _NCHUNK - 3), _NCHUNK):
        out_copy(i).wait()


@functools.partial(
    pl.kernel,
    out_type=jax.ShapeDtypeStruct((_SC_ROWS * _D,), jnp.float32),
    mesh=plsc.VectorSubcoreMesh(core_axis_name="c", subcore_axis_name="s"),
    scratch_types=[
        [pltpu.VMEM((_CD,), jnp.float32)] * _NBUF,
        [pltpu.SemaphoreType.DMA] * _NBUF,
        [pltpu.SemaphoreType.DMA] * _NBUF,
    ],
)
def _sc_copy(x_hbm, o_hbm, xbufs, xsems, osems):
    _sc_copy_body(x_hbm, o_hbm, xbufs, xsems, osems)


def kernel(x, pe_table):
    tc_out = _tc_add(x, pe_table)
    sc_out = _sc_copy(x.reshape(-1)[: _SC_ROWS * _D])
    return lax.optimization_barrier((tc_out, sc_out))[0]


# speedup vs baseline: 1.0032x; 1.0032x over previous
import jax
import jax.numpy as jnp
from jax.experimental import pallas as pl
from jax.experimental.pallas import tpu as pltpu

_BS = 2048


def _add_body(x_ref, pe_ref, o_ref):
    o_ref[...] = x_ref[...] + pe_ref[...][None]


def kernel(x, pe_table):
    B, S, D = x.shape
    grid = (S // _BS, B)
    return pl.pallas_call(
        _add_body,
        grid=grid,
        in_specs=[
            pl.BlockSpec((1, _BS, D), lambda s, b: (b, s, 0)),
            pl.BlockSpec((_BS, D), lambda s, b: (s, 0)),
        ],
        out_specs=pl.BlockSpec((1, _BS, D), lambda s, b: (b, s, 0)),
        out_shape=jax.ShapeDtypeStruct((B, S, D), x.dtype),
        compiler_params=pltpu.CompilerParams(
            dimension_semantics=("arbitrary", "arbitrary"),
        ),
    )(x, pe_table)
